# Initial kernel scaffold; baseline (speedup 1.0000x reference)
import jax
import jax.numpy as jnp
from jax.experimental import pallas as pl


def kernel(inp, input_grid, output_grid, nbr_idx, P_W0, P_b0, P_W1, P_b1, K_W0, K_b0, K_W1, K_b1, ln_g, ln_b):
    raise NotImplementedError("write your pallas kernel here")


# speedup vs baseline: 6.4677x; 6.4677x over previous
import functools

import jax
import jax.numpy as jnp
from jax import lax
from jax.experimental import pallas as pl
from jax.experimental.pallas import tpu as pltpu
from jax.experimental.pallas import tpu_sc as plsc

_SQRT_HALF = 0.7071067811865476


def _gelu(t):
    return 0.5 * t * (1.0 + lax.erf(t * _SQRT_HALF))


def _prep_body(inp_ref, gi_ref, go_ref, pw0_ref, pb0_ref, pw1_ref, pb1_ref,
               kw0a_ref, kw0b_ref, kw0f_ref, kb0_ref,
               x_ref, u_ref, g_ref):
    h = jnp.dot(inp_ref[...], pw0_ref[...], preferred_element_type=jnp.float32) + pb0_ref[...]
    x = jnp.dot(_gelu(h), pw1_ref[...], preferred_element_type=jnp.float32) + pb1_ref[...]
    x_ref[...] = x
    u_ref[...] = (jnp.dot(x, kw0f_ref[...], preferred_element_type=jnp.float32)
                  + jnp.dot(gi_ref[...], kw0a_ref[...], preferred_element_type=jnp.float32))
    g_ref[...] = (jnp.dot(go_ref[...], kw0b_ref[...], preferred_element_type=jnp.float32)
                  + kb0_ref[...])


def _post_body(rows_ref, g_ref, x_ref, kw1_ref, kb1_ref, lng_ref, lnb_ref, out_ref):
    d = g_ref.shape[-1]
    rows = rows_ref[...][:, :, :d]
    hidden = _gelu(rows + g_ref[...][None, :, :])
    s = jnp.mean(hidden, axis=0)
    o = (jnp.dot(s, kw1_ref[...], preferred_element_type=jnp.float32)
         + kb1_ref[...] + x_ref[...])
    mu = jnp.mean(o, axis=-1, keepdims=True)
    d = o - mu
    var = jnp.mean(d * d, axis=-1, keepdims=True)
    out_ref[...] = d * lax.rsqrt(var + 1e-5) * lng_ref[...] + lnb_ref[...]


def _sc_gather(table, idx2, n_rows, d):
    w = 256

    mesh = plsc.VectorSubcoreMesh(core_axis_name="c", subcore_axis_name="s")

    @functools.partial(
        pl.kernel,
        out_type=jax.ShapeDtypeStruct((n_rows, d), jnp.float32),
        mesh=mesh,
    )
    def k(table_hbm, idx_hbm, out_hbm):
        def body(i_vmem, o_vmem):
            pltpu.sync_copy(table_hbm.at[i_vmem.at[0]], o_vmem)

        pltpu.emit_pipeline(
            body,
            grid=(n_rows // w,),
            in_specs=[pl.BlockSpec((1, w), lambda i: (0, i))],
            out_specs=[pl.BlockSpec((w, d), lambda i: (i, 0))],
            core_axis_name=("c", "s"),
            dimension_semantics=(pltpu.PARALLEL,),
        )(idx_hbm, out_hbm)

    return k(table, idx2)


def kernel(inp, input_grid, output_grid, nbr_idx, P_W0, P_b0, P_W1, P_b1,
           K_W0, K_b0, K_W1, K_b1, ln_g, ln_b):
    B, N, _ = inp.shape
    D = P_W1.shape[1]
    K = nbr_idx.shape[1]
    nd = input_grid.shape[1]

    inp2 = inp.reshape(N, -1)
    pad = 8 - nd
    gi = jnp.pad(input_grid, ((0, 0), (0, pad)))
    go = jnp.pad(output_grid, ((0, 0), (0, pad)))
    DP = 128
    kw0a = jnp.pad(K_W0[:nd], ((0, pad), (0, DP - D)))
    kw0b = jnp.pad(K_W0[nd:2 * nd], ((0, pad), (0, 0)))
    kw0f = jnp.pad(K_W0[2 * nd:], ((0, 0), (0, DP - D)))

    x, u, g = pl.pallas_call(
        _prep_body,
        out_shape=[jax.ShapeDtypeStruct((N, D), jnp.float32),
                   jax.ShapeDtypeStruct((N, DP), jnp.float32),
                   jax.ShapeDtypeStruct((N, D), jnp.float32)],
    )(inp2, gi, go, P_W0, P_b0.reshape(1, -1), P_W1, P_b1.reshape(1, -1),
      kw0a, kw0b, kw0f, K_b0.reshape(1, -1))

    idx2 = nbr_idx.astype(jnp.int32).T.reshape(1, K * N)
    rows = _sc_gather(u, idx2, K * N, DP).reshape(K, N, DP)

    NB = 400
    out = pl.pallas_call(
        _post_body,
        grid=(N // NB,),
        in_specs=[
            pl.BlockSpec((K, NB, DP), lambda i: (0, i, 0)),
            pl.BlockSpec((NB, D), lambda i: (i, 0)),
            pl.BlockSpec((NB, D), lambda i: (i, 0)),
            pl.BlockSpec((D, D), lambda i: (0, 0)),
            pl.BlockSpec((1, D), lambda i: (0, 0)),
            pl.BlockSpec((1, D), lambda i: (0, 0)),
            pl.BlockSpec((1, D), lambda i: (0, 0)),
        ],
        out_specs=pl.BlockSpec((NB, D), lambda i: (i, 0)),
        out_shape=jax.ShapeDtypeStruct((N, D), jnp.float32),
    )(rows, g, x, K_W1, K_b1.reshape(1, -1), ln_g.reshape(1, -1), ln_b.reshape(1, -1))

    return out.reshape(B, N, D)
